# baseline (speedup 1.0000x reference)
import functools

import jax
import jax.numpy as jnp
from jax import lax
from jax.experimental import pallas as pl
from jax.experimental.pallas import tpu as pltpu
from jax.experimental.pallas import tpu_sc as plsc

_NUM_SENTENCES = 100000
_SEQ = 20
_DIM = 64

_NC = 2
_NS = 16
_NW = _NC * _NS
_LANES = 16

_G = 16
_NBUF = 2


def kernel(sentence_ids, dataset_ids, cache):
    batch = sentence_ids.shape[0]
    b_per_w = batch // _NW
    ngroup = b_per_w // _G

    mesh = plsc.VectorSubcoreMesh(
        core_axis_name="c", subcore_axis_name="s",
        num_cores=_NC, num_subcores=_NS)

    @functools.partial(
        pl.kernel,
        mesh=mesh,
        out_type=jax.ShapeDtypeStruct((batch, _DIM), jnp.float32),
        scratch_types=[
            pltpu.VMEM((b_per_w,), jnp.int32),
            pltpu.VMEM((b_per_w,), jnp.int32),
            pltpu.VMEM((_NBUF, _G, _SEQ, _DIM), jnp.float32),
            pltpu.VMEM((_G, _DIM), jnp.float32),
            pltpu.SemaphoreType.DMA,
        ],
    )
    def sc_kernel(sid_hbm, did_hbm, cache_hbm, out_hbm,
                  ids_v, dids_v, rows_v, out_v, sem):
        wid = lax.axis_index("s") * _NC + lax.axis_index("c")
        base = wid * b_per_w

        pltpu.sync_copy(sid_hbm.at[pl.ds(base, b_per_w)], ids_v)
        pltpu.sync_copy(did_hbm.at[pl.ds(base, b_per_w)], dids_v)
        for j in range(b_per_w // _LANES):
            sl = pl.ds(j * _LANES, _LANES)
            ids_v[sl] = ids_v[sl] + dids_v[sl] * _NUM_SENTENCES

        def issue_group(g, buf):
            v = ids_v[pl.ds(pl.multiple_of(g * _G, _G), _G)]
            for l in range(_G):
                pltpu.async_copy(cache_hbm.at[v[l]], rows_v.at[buf, l], sem)

        issue_group(0, 0)
        issue_group(1, 1)

        def group_body(g, carry):
            buf = lax.rem(g, _NBUF)
            pltpu.make_async_copy(
                cache_hbm.at[pl.ds(0, _G)], rows_v.at[buf], sem).wait()

            def pool_body(s, carry2):
                for d in range(_DIM // _LANES):
                    acc = rows_v[buf, s, 0, pl.ds(d * _LANES, _LANES)]
                    for t in range(1, _SEQ):
                        acc = acc + rows_v[buf, s, t, pl.ds(d * _LANES,
                                                            _LANES)]
                    out_v[s, pl.ds(d * _LANES, _LANES)] = acc * (1.0 / _SEQ)
                return carry2

            lax.fori_loop(0, _G, pool_body, 0, unroll=False)
            off = pl.multiple_of(base + g * _G, _G)
            pltpu.sync_copy(out_v, out_hbm.at[pl.ds(off, _G)])

            @pl.when(g + _NBUF < ngroup)
            def _():
                issue_group(g + _NBUF, buf)
            return carry

        lax.fori_loop(0, ngroup, group_body, 0, unroll=False)

    return sc_kernel(sentence_ids, dataset_ids, cache)


# speedup vs baseline: 4.1994x; 4.1994x over previous
import functools

import jax
import jax.numpy as jnp
from jax import lax
from jax.experimental import pallas as pl
from jax.experimental.pallas import tpu as pltpu
from jax.experimental.pallas import tpu_sc as plsc

_NUM_SENTENCES = 100000
_SEQ = 20
_DIM = 64

_NC = 2
_NS = 16
_NW = _NC * _NS
_LANES = 16

_BS = 2048
_GI = 128
_PDIM = 128


def _pool_body(ct_ref, pooled_ref):
    x = ct_ref[...]
    s = jnp.sum(x, axis=0) * (1.0 / _SEQ)
    eye = jnp.asarray(
        lax.broadcasted_iota(jnp.int32, (_DIM, _PDIM), 0)
        == lax.broadcasted_iota(jnp.int32, (_DIM, _PDIM), 1),
        dtype=jnp.float32)
    pooled_ref[...] = lax.dot_general(
        s, eye, (((0,), (0,)), ((), ())),
        preferred_element_type=jnp.float32)


def _tc_pool(ct):
    nblk = (_NUM_SENTENCES + _BS - 1) // _BS
    return pl.pallas_call(
        _pool_body,
        grid=(nblk,),
        in_specs=[pl.BlockSpec((_SEQ, _DIM, _BS), lambda i: (0, 0, i))],
        out_specs=pl.BlockSpec((_BS, _PDIM), lambda i: (i, 0)),
        out_shape=jax.ShapeDtypeStruct((_NUM_SENTENCES, _PDIM), jnp.float32),
    )(ct)


def kernel(sentence_ids, dataset_ids, cache):
    batch = sentence_ids.shape[0]
    b_per_w = batch // _NW

    ct = cache.transpose(1, 2, 0)
    pooled = _tc_pool(ct)

    mesh = plsc.VectorSubcoreMesh(
        core_axis_name="c", subcore_axis_name="s",
        num_cores=_NC, num_subcores=_NS)

    @functools.partial(
        pl.kernel,
        mesh=mesh,
        out_type=jax.ShapeDtypeStruct((batch, _DIM), jnp.float32),
        scratch_types=[
            pltpu.VMEM((b_per_w,), jnp.int32),
            pltpu.VMEM((b_per_w,), jnp.int32),
            pltpu.VMEM((b_per_w // _GI, _GI), jnp.int32),
            pltpu.VMEM((2, _GI, _PDIM), jnp.float32),
            pltpu.VMEM((_GI, _DIM), jnp.float32),
            pltpu.SemaphoreType.DMA,
        ],
    )
    def sc_gather(sid_hbm, did_hbm, pooled_hbm, out_hbm,
                  ids_v, dids_v, idx_v, rows_v, out_v, sem):
        wid = lax.axis_index("s") * _NC + lax.axis_index("c")
        base = wid * b_per_w
        nch = b_per_w // _GI

        pltpu.sync_copy(sid_hbm.at[pl.ds(base, b_per_w)], ids_v)
        pltpu.sync_copy(did_hbm.at[pl.ds(base, b_per_w)], dids_v)
        for j in range(b_per_w // _LANES):
            sl = pl.ds(j * _LANES, _LANES)
            g, h = divmod(j * _LANES, _GI)
            idx_v[g, pl.ds(h, _LANES)] = (
                ids_v[sl] + dids_v[sl] * _NUM_SENTENCES)

        def issue(g):
            pltpu.async_copy(
                pooled_hbm.at[idx_v.at[g]], rows_v.at[g % 2], sem)

        issue(0)
        issue(1)
        for g in range(nch):
            buf = g % 2
            pltpu.make_async_copy(
                pooled_hbm.at[pl.ds(0, _GI)], rows_v.at[buf], sem).wait()

            def compact_body(s, carry, _buf=buf):
                for d in range(_DIM // _LANES):
                    sl = pl.ds(d * _LANES, _LANES)
                    out_v[s, sl] = rows_v[_buf, s, sl]
                return carry

            lax.fori_loop(0, _GI, compact_body, 0, unroll=False)
            pltpu.sync_copy(out_v, out_hbm.at[pl.ds(base + g * _GI, _GI)])
            if g + 2 < nch:
                issue(g + 2)

    return sc_gather(sentence_ids, dataset_ids, pooled)


# speedup vs baseline: 4.2287x; 1.0070x over previous
import functools

import jax
import jax.numpy as jnp
from jax import lax
from jax.experimental import pallas as pl
from jax.experimental.pallas import tpu as pltpu
from jax.experimental.pallas import tpu_sc as plsc

_NUM_SENTENCES = 100000
_SEQ = 20
_DIM = 64

_NC = 2
_NS = 16
_NW = _NC * _NS
_LANES = 16

_BS = 4096
_GI = 128
_PDIM = 128


def _pool_body(ct_ref, pooled_ref):
    x = ct_ref[...]
    s = jnp.sum(x, axis=0) * (1.0 / _SEQ)
    eye = jnp.asarray(
        lax.broadcasted_iota(jnp.int32, (_DIM, _PDIM), 0)
        == lax.broadcasted_iota(jnp.int32, (_DIM, _PDIM), 1),
        dtype=jnp.float32)
    pooled_ref[...] = lax.dot_general(
        s, eye, (((0,), (0,)), ((), ())),
        preferred_element_type=jnp.float32)


def _tc_pool(ct):
    nblk = (_NUM_SENTENCES + _BS - 1) // _BS
    return pl.pallas_call(
        _pool_body,
        grid=(nblk,),
        in_specs=[pl.BlockSpec((_SEQ, _DIM, _BS), lambda i: (0, 0, i))],
        out_specs=pl.BlockSpec((_BS, _PDIM), lambda i: (i, 0)),
        out_shape=jax.ShapeDtypeStruct((_NUM_SENTENCES, _PDIM), jnp.float32),
    )(ct)


def kernel(sentence_ids, dataset_ids, cache):
    batch = sentence_ids.shape[0]
    b_per_w = batch // _NW

    ct = cache.transpose(1, 2, 0)
    pooled = _tc_pool(ct)

    mesh = plsc.VectorSubcoreMesh(
        core_axis_name="c", subcore_axis_name="s",
        num_cores=_NC, num_subcores=_NS)

    @functools.partial(
        pl.kernel,
        mesh=mesh,
        out_type=jax.ShapeDtypeStruct((batch, _DIM), jnp.float32),
        scratch_types=[
            pltpu.VMEM((b_per_w,), jnp.int32),
            pltpu.VMEM((b_per_w,), jnp.int32),
            pltpu.VMEM((b_per_w // _GI, _GI), jnp.int32),
            pltpu.VMEM((2, _GI, _PDIM), jnp.float32),
            pltpu.VMEM((_GI, _DIM), jnp.float32),
            pltpu.SemaphoreType.DMA,
        ],
    )
    def sc_gather(sid_hbm, did_hbm, pooled_hbm, out_hbm,
                  ids_v, dids_v, idx_v, rows_v, out_v, sem):
        wid = lax.axis_index("s") * _NC + lax.axis_index("c")
        base = wid * b_per_w
        nch = b_per_w // _GI

        pltpu.sync_copy(sid_hbm.at[pl.ds(base, b_per_w)], ids_v)
        pltpu.sync_copy(did_hbm.at[pl.ds(base, b_per_w)], dids_v)
        for j in range(b_per_w // _LANES):
            sl = pl.ds(j * _LANES, _LANES)
            g, h = divmod(j * _LANES, _GI)
            idx_v[g, pl.ds(h, _LANES)] = (
                ids_v[sl] + dids_v[sl] * _NUM_SENTENCES)

        def issue(g):
            pltpu.async_copy(
                pooled_hbm.at[idx_v.at[g]], rows_v.at[g % 2], sem)

        issue(0)
        issue(1)
        for g in range(nch):
            buf = g % 2
            pltpu.make_async_copy(
                pooled_hbm.at[pl.ds(0, _GI)], rows_v.at[buf], sem).wait()

            def compact_body(s, carry, _buf=buf):
                for d in range(_DIM // _LANES):
                    sl = pl.ds(d * _LANES, _LANES)
                    out_v[s, sl] = rows_v[_buf, s, sl]
                return carry

            lax.fori_loop(0, _GI, compact_body, 0, unroll=False)
            pltpu.sync_copy(out_v, out_hbm.at[pl.ds(base + g * _GI, _GI)])
            if g + 2 < nch:
                issue(g + 2)

    return sc_gather(sentence_ids, dataset_ids, pooled)


# speedup vs baseline: 4.3699x; 1.0334x over previous
import functools

import jax
import jax.numpy as jnp
from jax import lax
from jax.experimental import pallas as pl
from jax.experimental.pallas import tpu as pltpu
from jax.experimental.pallas import tpu_sc as plsc

_NUM_SENTENCES = 100000
_SEQ = 20
_DIM = 64

_NC = 2
_NS = 16
_NW = _NC * _NS
_LANES = 16

_BS = 5120
_GI = 128
_PDIM = 128


def _pool_body(ct_ref, pooled_ref):
    x = ct_ref[...]
    s = jnp.sum(x, axis=0) * (1.0 / _SEQ)
    eye = jnp.asarray(
        lax.broadcasted_iota(jnp.int32, (_DIM, _PDIM), 0)
        == lax.broadcasted_iota(jnp.int32, (_DIM, _PDIM), 1),
        dtype=jnp.float32)
    pooled_ref[...] = lax.dot_general(
        s, eye, (((0,), (0,)), ((), ())),
        preferred_element_type=jnp.float32)


def _tc_pool(ct):
    nblk = (_NUM_SENTENCES + _BS - 1) // _BS
    return pl.pallas_call(
        _pool_body,
        grid=(nblk,),
        in_specs=[pl.BlockSpec((_SEQ, _DIM, _BS), lambda i: (0, 0, i))],
        out_specs=pl.BlockSpec((_BS, _PDIM), lambda i: (i, 0)),
        out_shape=jax.ShapeDtypeStruct((_NUM_SENTENCES, _PDIM), jnp.float32),
    )(ct)


def kernel(sentence_ids, dataset_ids, cache):
    batch = sentence_ids.shape[0]
    b_per_w = batch // _NW

    ct = cache.transpose(1, 2, 0)
    pooled = _tc_pool(ct)

    mesh = plsc.VectorSubcoreMesh(
        core_axis_name="c", subcore_axis_name="s",
        num_cores=_NC, num_subcores=_NS)

    @functools.partial(
        pl.kernel,
        mesh=mesh,
        out_type=jax.ShapeDtypeStruct((batch, _DIM), jnp.float32),
        scratch_types=[
            pltpu.VMEM((b_per_w,), jnp.int32),
            pltpu.VMEM((b_per_w,), jnp.int32),
            pltpu.VMEM((b_per_w // _GI, _GI), jnp.int32),
            pltpu.VMEM((2, _GI, _PDIM), jnp.float32),
            pltpu.VMEM((_GI, _DIM), jnp.float32),
            pltpu.SemaphoreType.DMA,
        ],
    )
    def sc_gather(sid_hbm, did_hbm, pooled_hbm, out_hbm,
                  ids_v, dids_v, idx_v, rows_v, out_v, sem):
        wid = lax.axis_index("s") * _NC + lax.axis_index("c")
        base = wid * b_per_w
        nch = b_per_w // _GI

        pltpu.sync_copy(sid_hbm.at[pl.ds(base, b_per_w)], ids_v)
        pltpu.sync_copy(did_hbm.at[pl.ds(base, b_per_w)], dids_v)
        for j in range(b_per_w // _LANES):
            sl = pl.ds(j * _LANES, _LANES)
            g, h = divmod(j * _LANES, _GI)
            idx_v[g, pl.ds(h, _LANES)] = (
                ids_v[sl] + dids_v[sl] * _NUM_SENTENCES)

        def issue(g):
            pltpu.async_copy(
                pooled_hbm.at[idx_v.at[g]], rows_v.at[g % 2], sem)

        issue(0)
        issue(1)
        for g in range(nch):
            buf = g % 2
            pltpu.make_async_copy(
                pooled_hbm.at[pl.ds(0, _GI)], rows_v.at[buf], sem).wait()

            def compact_body(s, carry, _buf=buf):
                for d in range(_DIM // _LANES):
                    sl = pl.ds(d * _LANES, _LANES)
                    out_v[s, sl] = rows_v[_buf, s, sl]
                return carry

            lax.fori_loop(0, _GI, compact_body, 0, unroll=False)
            pltpu.sync_copy(out_v, out_hbm.at[pl.ds(base + g * _GI, _GI)])
            if g + 2 < nch:
                issue(g + 2)

    return sc_gather(sentence_ids, dataset_ids, pooled)


# speedup vs baseline: 4.5521x; 1.0417x over previous
import functools

import jax
import jax.numpy as jnp
from jax import lax
from jax.experimental import pallas as pl
from jax.experimental.pallas import tpu as pltpu
from jax.experimental.pallas import tpu_sc as plsc

_NUM_SENTENCES = 100000
_SEQ = 20
_DIM = 64

_NC = 2
_NS = 16
_NW = _NC * _NS
_LANES = 16

_BS = 5504
_GI = 128
_PDIM = 128


def _pool_body(ct_ref, pooled_ref):
    x = ct_ref[...]
    s = jnp.sum(x, axis=0) * (1.0 / _SEQ)
    eye = jnp.asarray(
        lax.broadcasted_iota(jnp.int32, (_DIM, _PDIM), 0)
        == lax.broadcasted_iota(jnp.int32, (_DIM, _PDIM), 1),
        dtype=jnp.float32)
    pooled_ref[...] = lax.dot_general(
        s, eye, (((0,), (0,)), ((), ())),
        preferred_element_type=jnp.float32)


def _tc_pool(ct):
    nblk = (_NUM_SENTENCES + _BS - 1) // _BS
    return pl.pallas_call(
        _pool_body,
        grid=(nblk,),
        in_specs=[pl.BlockSpec((_SEQ, _DIM, _BS), lambda i: (0, 0, i))],
        out_specs=pl.BlockSpec((_BS, _PDIM), lambda i: (i, 0)),
        out_shape=jax.ShapeDtypeStruct((_NUM_SENTENCES, _PDIM), jnp.float32),
        compiler_params=pltpu.CompilerParams(vmem_limit_bytes=64 * 1024 * 1024),
    )(ct)


def kernel(sentence_ids, dataset_ids, cache):
    batch = sentence_ids.shape[0]
    b_per_w = batch // _NW

    ct = cache.transpose(1, 2, 0)
    pooled = _tc_pool(ct)

    mesh = plsc.VectorSubcoreMesh(
        core_axis_name="c", subcore_axis_name="s",
        num_cores=_NC, num_subcores=_NS)

    @functools.partial(
        pl.kernel,
        mesh=mesh,
        out_type=jax.ShapeDtypeStruct((batch, _DIM), jnp.float32),
        scratch_types=[
            pltpu.VMEM((b_per_w,), jnp.int32),
            pltpu.VMEM((b_per_w,), jnp.int32),
            pltpu.VMEM((b_per_w // _GI, _GI), jnp.int32),
            pltpu.VMEM((2, _GI, _PDIM), jnp.float32),
            pltpu.VMEM((_GI, _DIM), jnp.float32),
            pltpu.SemaphoreType.DMA,
        ],
    )
    def sc_gather(sid_hbm, did_hbm, pooled_hbm, out_hbm,
                  ids_v, dids_v, idx_v, rows_v, out_v, sem):
        wid = lax.axis_index("s") * _NC + lax.axis_index("c")
        base = wid * b_per_w
        nch = b_per_w // _GI

        pltpu.sync_copy(sid_hbm.at[pl.ds(base, b_per_w)], ids_v)
        pltpu.sync_copy(did_hbm.at[pl.ds(base, b_per_w)], dids_v)
        for j in range(b_per_w // _LANES):
            sl = pl.ds(j * _LANES, _LANES)
            g, h = divmod(j * _LANES, _GI)
            idx_v[g, pl.ds(h, _LANES)] = (
                ids_v[sl] + dids_v[sl] * _NUM_SENTENCES)

        def issue(g):
            pltpu.async_copy(
                pooled_hbm.at[idx_v.at[g]], rows_v.at[g % 2], sem)

        issue(0)
        issue(1)
        for g in range(nch):
            buf = g % 2
            pltpu.make_async_copy(
                pooled_hbm.at[pl.ds(0, _GI)], rows_v.at[buf], sem).wait()

            def compact_body(s, carry, _buf=buf):
                for d in range(_DIM // _LANES):
                    sl = pl.ds(d * _LANES, _LANES)
                    out_v[s, sl] = rows_v[_buf, s, sl]
                return carry

            lax.fori_loop(0, _GI, compact_body, 0, unroll=False)
            pltpu.sync_copy(out_v, out_hbm.at[pl.ds(base + g * _GI, _GI)])
            if g + 2 < nch:
                issue(g + 2)

    return sc_gather(sentence_ids, dataset_ids, pooled)


# speedup vs baseline: 4.5637x; 1.0026x over previous
import functools

import jax
import jax.numpy as jnp
from jax import lax
from jax.experimental import pallas as pl
from jax.experimental.pallas import tpu as pltpu
from jax.experimental.pallas import tpu_sc as plsc

_NUM_SENTENCES = 100000
_SEQ = 20
_DIM = 64

_NC = 2
_NS = 16
_NW = _NC * _NS
_LANES = 16

_BS = 5376
_GI = 128
_PDIM = 128


def _pool_body(ct_ref, pooled_ref):
    x = ct_ref[...]
    s = jnp.sum(x, axis=0) * (1.0 / _SEQ)
    eye = jnp.asarray(
        lax.broadcasted_iota(jnp.int32, (_DIM, _PDIM), 0)
        == lax.broadcasted_iota(jnp.int32, (_DIM, _PDIM), 1),
        dtype=jnp.float32)
    pooled_ref[...] = lax.dot_general(
        s, eye, (((0,), (0,)), ((), ())),
        preferred_element_type=jnp.float32)


def _tc_pool(ct):
    nblk = (_NUM_SENTENCES + _BS - 1) // _BS
    return pl.pallas_call(
        _pool_body,
        grid=(nblk,),
        in_specs=[pl.BlockSpec((_SEQ, _DIM, _BS), lambda i: (0, 0, i))],
        out_specs=pl.BlockSpec((_BS, _PDIM), lambda i: (i, 0)),
        out_shape=jax.ShapeDtypeStruct((_NUM_SENTENCES, _PDIM), jnp.float32),
        compiler_params=pltpu.CompilerParams(vmem_limit_bytes=64 * 1024 * 1024),
    )(ct)


def kernel(sentence_ids, dataset_ids, cache):
    batch = sentence_ids.shape[0]
    b_per_w = batch // _NW

    ct = cache.transpose(1, 2, 0)
    pooled = _tc_pool(ct)

    mesh = plsc.VectorSubcoreMesh(
        core_axis_name="c", subcore_axis_name="s",
        num_cores=_NC, num_subcores=_NS)

    @functools.partial(
        pl.kernel,
        mesh=mesh,
        out_type=jax.ShapeDtypeStruct((batch, _DIM), jnp.float32),
        scratch_types=[
            pltpu.VMEM((b_per_w,), jnp.int32),
            pltpu.VMEM((b_per_w,), jnp.int32),
            pltpu.VMEM((b_per_w // _GI, _GI), jnp.int32),
            pltpu.VMEM((2, _GI, _PDIM), jnp.float32),
            pltpu.VMEM((_GI, _DIM), jnp.float32),
            pltpu.SemaphoreType.DMA,
        ],
    )
    def sc_gather(sid_hbm, did_hbm, pooled_hbm, out_hbm,
                  ids_v, dids_v, idx_v, rows_v, out_v, sem):
        wid = lax.axis_index("s") * _NC + lax.axis_index("c")
        base = wid * b_per_w
        nch = b_per_w // _GI

        pltpu.sync_copy(sid_hbm.at[pl.ds(base, b_per_w)], ids_v)
        pltpu.sync_copy(did_hbm.at[pl.ds(base, b_per_w)], dids_v)
        for j in range(b_per_w // _LANES):
            sl = pl.ds(j * _LANES, _LANES)
            g, h = divmod(j * _LANES, _GI)
            idx_v[g, pl.ds(h, _LANES)] = (
                ids_v[sl] + dids_v[sl] * _NUM_SENTENCES)

        def issue(g):
            pltpu.async_copy(
                pooled_hbm.at[idx_v.at[g]], rows_v.at[g % 2], sem)

        issue(0)
        issue(1)
        for g in range(nch):
            buf = g % 2
            pltpu.make_async_copy(
                pooled_hbm.at[pl.ds(0, _GI)], rows_v.at[buf], sem).wait()

            def compact_body(s, carry, _buf=buf):
                for d in range(_DIM // _LANES):
                    sl = pl.ds(d * _LANES, _LANES)
                    out_v[s, sl] = rows_v[_buf, s, sl]
                return carry

            lax.fori_loop(0, _GI, compact_body, 0, unroll=False)
            pltpu.sync_copy(out_v, out_hbm.at[pl.ds(base + g * _GI, _GI)])
            if g + 2 < nch:
                issue(g + 2)

    return sc_gather(sentence_ids, dataset_ids, pooled)


# speedup vs baseline: 4.5647x; 1.0002x over previous
import functools

import jax
import jax.numpy as jnp
from jax import lax
from jax.experimental import pallas as pl
from jax.experimental.pallas import tpu as pltpu
from jax.experimental.pallas import tpu_sc as plsc

_NUM_SENTENCES = 100000
_SEQ = 20
_DIM = 64

_NC = 2
_NS = 16
_NW = _NC * _NS
_LANES = 16

_BS = 3584
_GI = 128
_PDIM = 128


def _pool_body(ct_ref, pooled_ref):
    x = ct_ref[...]
    s = jnp.sum(x, axis=0) * (1.0 / _SEQ)
    eye = jnp.asarray(
        lax.broadcasted_iota(jnp.int32, (_DIM, _PDIM), 0)
        == lax.broadcasted_iota(jnp.int32, (_DIM, _PDIM), 1),
        dtype=jnp.float32)
    pooled_ref[...] = lax.dot_general(
        s, eye, (((0,), (0,)), ((), ())),
        preferred_element_type=jnp.float32)


def _tc_pool(ct):
    nblk = (_NUM_SENTENCES + _BS - 1) // _BS
    return pl.pallas_call(
        _pool_body,
        grid=(nblk,),
        in_specs=[pl.BlockSpec((_SEQ, _DIM, _BS), lambda i: (0, 0, i))],
        out_specs=pl.BlockSpec((_BS, _PDIM), lambda i: (i, 0)),
        out_shape=jax.ShapeDtypeStruct((_NUM_SENTENCES, _PDIM), jnp.float32),
        compiler_params=pltpu.CompilerParams(vmem_limit_bytes=64 * 1024 * 1024),
    )(ct)


def kernel(sentence_ids, dataset_ids, cache):
    batch = sentence_ids.shape[0]
    b_per_w = batch // _NW

    ct = cache.transpose(1, 2, 0)
    pooled = _tc_pool(ct)

    mesh = plsc.VectorSubcoreMesh(
        core_axis_name="c", subcore_axis_name="s",
        num_cores=_NC, num_subcores=_NS)

    @functools.partial(
        pl.kernel,
        mesh=mesh,
        out_type=jax.ShapeDtypeStruct((batch, _DIM), jnp.float32),
        scratch_types=[
            pltpu.VMEM((b_per_w,), jnp.int32),
            pltpu.VMEM((b_per_w,), jnp.int32),
            pltpu.VMEM((b_per_w // _GI, _GI), jnp.int32),
            pltpu.VMEM((2, _GI, _PDIM), jnp.float32),
            pltpu.VMEM((_GI, _DIM), jnp.float32),
            pltpu.SemaphoreType.DMA,
        ],
    )
    def sc_gather(sid_hbm, did_hbm, pooled_hbm, out_hbm,
                  ids_v, dids_v, idx_v, rows_v, out_v, sem):
        wid = lax.axis_index("s") * _NC + lax.axis_index("c")
        base = wid * b_per_w
        nch = b_per_w // _GI

        pltpu.sync_copy(sid_hbm.at[pl.ds(base, b_per_w)], ids_v)
        pltpu.sync_copy(did_hbm.at[pl.ds(base, b_per_w)], dids_v)
        for j in range(b_per_w // _LANES):
            sl = pl.ds(j * _LANES, _LANES)
            g, h = divmod(j * _LANES, _GI)
            idx_v[g, pl.ds(h, _LANES)] = (
                ids_v[sl] + dids_v[sl] * _NUM_SENTENCES)

        def issue(g):
            pltpu.async_copy(
                pooled_hbm.at[idx_v.at[g]], rows_v.at[g % 2], sem)

        issue(0)
        issue(1)
        for g in range(nch):
            buf = g % 2
            pltpu.make_async_copy(
                pooled_hbm.at[pl.ds(0, _GI)], rows_v.at[buf], sem).wait()

            def compact_body(s, carry, _buf=buf):
                for d in range(_DIM // _LANES):
                    sl = pl.ds(d * _LANES, _LANES)
                    out_v[s, sl] = rows_v[_buf, s, sl]
                return carry

            lax.fori_loop(0, _GI, compact_body, 0, unroll=False)
            pltpu.sync_copy(out_v, out_hbm.at[pl.ds(base + g * _GI, _GI)])
            if g + 2 < nch:
                issue(g + 2)

    return sc_gather(sentence_ids, dataset_ids, pooled)


# speedup vs baseline: 4.5769x; 1.0027x over previous
import functools

import jax
import jax.numpy as jnp
from jax import lax
from jax.experimental import pallas as pl
from jax.experimental.pallas import tpu as pltpu
from jax.experimental.pallas import tpu_sc as plsc

_NUM_SENTENCES = 100000
_SEQ = 20
_DIM = 64

_NC = 2
_NS = 16
_NW = _NC * _NS
_LANES = 16

_BS = 3584
_GI = 128
_PDIM = 128


def _pool_body(ct_ref, pooled_ref):
    x = ct_ref[...]
    s = jnp.sum(x, axis=0) * (1.0 / _SEQ)
    t = jnp.transpose(s, (1, 0))
    pooled_ref[...] = jnp.concatenate(
        [t, jnp.zeros((t.shape[0], _PDIM - _DIM), jnp.float32)], axis=1)


def _tc_pool(ct):
    nblk = (_NUM_SENTENCES + _BS - 1) // _BS
    return pl.pallas_call(
        _pool_body,
        grid=(nblk,),
        in_specs=[pl.BlockSpec((_SEQ, _DIM, _BS), lambda i: (0, 0, i))],
        out_specs=pl.BlockSpec((_BS, _PDIM), lambda i: (i, 0)),
        out_shape=jax.ShapeDtypeStruct((_NUM_SENTENCES, _PDIM), jnp.float32),
    )(ct)


def kernel(sentence_ids, dataset_ids, cache):
    batch = sentence_ids.shape[0]
    b_per_w = batch // _NW

    ct = cache.transpose(1, 2, 0)
    pooled = _tc_pool(ct)

    mesh = plsc.VectorSubcoreMesh(
        core_axis_name="c", subcore_axis_name="s",
        num_cores=_NC, num_subcores=_NS)

    @functools.partial(
        pl.kernel,
        mesh=mesh,
        out_type=jax.ShapeDtypeStruct((batch, _DIM), jnp.float32),
        scratch_types=[
            pltpu.VMEM((b_per_w,), jnp.int32),
            pltpu.VMEM((b_per_w,), jnp.int32),
            pltpu.VMEM((b_per_w // _GI, _GI), jnp.int32),
            pltpu.VMEM((2, _GI, _PDIM), jnp.float32),
            pltpu.VMEM((_GI, _DIM), jnp.float32),
            pltpu.SemaphoreType.DMA,
        ],
    )
    def sc_gather(sid_hbm, did_hbm, pooled_hbm, out_hbm,
                  ids_v, dids_v, idx_v, rows_v, out_v, sem):
        wid = lax.axis_index("s") * _NC + lax.axis_index("c")
        base = wid * b_per_w
        nch = b_per_w // _GI

        pltpu.sync_copy(sid_hbm.at[pl.ds(base, b_per_w)], ids_v)
        pltpu.sync_copy(did_hbm.at[pl.ds(base, b_per_w)], dids_v)
        for j in range(b_per_w // _LANES):
            sl = pl.ds(j * _LANES, _LANES)
            g, h = divmod(j * _LANES, _GI)
            idx_v[g, pl.ds(h, _LANES)] = (
                ids_v[sl] + dids_v[sl] * _NUM_SENTENCES)

        def issue(g):
            pltpu.async_copy(
                pooled_hbm.at[idx_v.at[g]], rows_v.at[g % 2], sem)

        issue(0)
        issue(1)
        for g in range(nch):
            buf = g % 2
            pltpu.make_async_copy(
                pooled_hbm.at[pl.ds(0, _GI)], rows_v.at[buf], sem).wait()

            def compact_body(s, carry, _buf=buf):
                for d in range(_DIM // _LANES):
                    sl = pl.ds(d * _LANES, _LANES)
                    out_v[s, sl] = rows_v[_buf, s, sl]
                return carry

            lax.fori_loop(0, _GI, compact_body, 0, unroll=False)
            pltpu.sync_copy(out_v, out_hbm.at[pl.ds(base + g * _GI, _GI)])
            if g + 2 < nch:
                issue(g + 2)

    return sc_gather(sentence_ids, dataset_ids, pooled)
